# Initial kernel scaffold; baseline (speedup 1.0000x reference)
import jax
import jax.numpy as jnp
from jax.experimental import pallas as pl


def kernel(seq_len, table):
    raise NotImplementedError("write your pallas kernel here")


# speedup vs baseline: 36.4536x; 36.4536x over previous
import jax
import jax.numpy as jnp
from jax.experimental import pallas as pl
from jax.experimental.pallas import tpu as pltpu

_MAXD = 128
_H = 16
_S = 2048
_B = 256
_EXT = 1024
_NB = _S // _B


def _rpb_kernel(tab_ref, out_ref, z_ref):
    I = pl.program_id(1)
    J = pl.program_id(2)

    t_low = tab_ref[0, 0, 0]
    t_high = tab_ref[0, 0, 2 * _MAXD]

    @pl.when(jnp.logical_and(I == 0, J == 0))
    def _build_master():
        p = jax.lax.broadcasted_iota(jnp.int32, (1, _EXT), 1)
        tabrow = tab_ref[0, 0:1, :]
        big = jnp.concatenate(
            [jnp.full((1, 384), t_low, jnp.float32),
             tabrow,
             jnp.full((1, 128), 0.0, jnp.float32)], axis=1)
        e = jnp.where(p > 640, t_high, big)
        y = jnp.broadcast_to(e, (_B, _EXT))
        il = jax.lax.broadcasted_iota(jnp.int32, (_B, _EXT), 0)
        for b in range(8):
            s = 1 << b
            rolled = jnp.concatenate([y[:, _EXT - s:], y[:, :_EXT - s]],
                                     axis=1)
            y = jnp.where((il & s) != 0, rolled, y)
        z_ref[...] = y

    d = J - I

    @pl.when(d < -1)
    def _():
        out_ref[0, 0] = jnp.full((_B, _B), t_low, jnp.float32)

    @pl.when(d > 1)
    def _():
        out_ref[0, 0] = jnp.full((_B, _B), t_high, jnp.float32)

    @pl.when(d == -1)
    def _():
        out_ref[0, 0] = z_ref[:, 256:512]

    @pl.when(d == 0)
    def _():
        out_ref[0, 0] = z_ref[:, 512:768]

    @pl.when(d == 1)
    def _():
        out_ref[0, 0] = z_ref[:, 768:1024]


def kernel(seq_len, table):
    tableT = jnp.zeros((_H, 1, 512), jnp.float32).at[:, 0, :2 * _MAXD + 1].set(
        table.T.astype(jnp.float32))
    out = pl.pallas_call(
        _rpb_kernel,
        grid=(_H, _NB, _NB),
        in_specs=[pl.BlockSpec((1, 1, 512), lambda h, i, j: (h, 0, 0))],
        out_specs=pl.BlockSpec((1, 1, _B, _B), lambda h, i, j: (0, h, i, j)),
        out_shape=jax.ShapeDtypeStruct((1, _H, _S, _S), jnp.float32),
        scratch_shapes=[pltpu.VMEM((_B, _EXT), jnp.float32)],
        compiler_params=pltpu.CompilerParams(
            dimension_semantics=("arbitrary", "arbitrary", "arbitrary")),
    )(tableT)
    return out


# speedup vs baseline: 190.6777x; 5.2307x over previous
import jax
import jax.numpy as jnp
from jax.experimental import pallas as pl
from jax.experimental.pallas import tpu as pltpu

_MAXD = 128
_H = 16
_S = 2048
_B = 256
_EXT = 512
_NB = _S // _B


def _rpb_kernel(tab_ref, out_ref, z_ref):
    t_low = tab_ref[0, 0, 0]
    t_high = tab_ref[0, 0, 2 * _MAXD]

    p = jax.lax.broadcasted_iota(jnp.int32, (1, _EXT), 1)
    tabrow = tab_ref[0, 0:1, :]
    big = jnp.concatenate(
        [jnp.full((1, 128), t_low, jnp.float32), tabrow[:, 0:384]], axis=1)
    e2 = jnp.where(p >= 384, t_high, big)
    y = jnp.broadcast_to(e2, (128, _EXT))
    il = jax.lax.broadcasted_iota(jnp.int32, (128, _EXT), 0)
    for b in range(7):
        s = 1 << b
        rolled = jnp.concatenate([y[:, _EXT - s:], y[:, :_EXT - s]], axis=1)
        y = jnp.where((il & s) != 0, rolled, y)
    z_ref[...] = y

    a_m1 = z_ref[:, 128:256]
    a_0 = z_ref[:, 256:384]
    a_p1 = z_ref[:, 384:512]

    l128 = jnp.full((128, 128), t_low, jnp.float32)
    h128 = jnp.full((128, 128), t_high, jnp.float32)
    low_t = jnp.full((_B, _B), t_low, jnp.float32)
    high_t = jnp.full((_B, _B), t_high, jnp.float32)

    t_0 = jnp.concatenate(
        [jnp.concatenate([a_0, a_p1], axis=1),
         jnp.concatenate([a_m1, a_0], axis=1)], axis=0)
    t_p1 = jnp.concatenate(
        [jnp.concatenate([h128, h128], axis=1),
         jnp.concatenate([a_p1, h128], axis=1)], axis=0)
    t_m1 = jnp.concatenate(
        [jnp.concatenate([l128, a_m1], axis=1),
         jnp.concatenate([l128, l128], axis=1)], axis=0)

    for ti in range(_NB):
        for tc in range(_NB):
            dt = tc - ti
            if dt <= -2:
                val = low_t
            elif dt == -1:
                val = t_m1
            elif dt == 0:
                val = t_0
            elif dt == 1:
                val = t_p1
            else:
                val = high_t
            out_ref[0, 0, ti * _B:(ti + 1) * _B, tc * _B:(tc + 1) * _B] = val


def kernel(seq_len, table):
    tableT = jnp.zeros((_H, 1, 512), jnp.float32).at[:, 0, :2 * _MAXD + 1].set(
        table.T.astype(jnp.float32))
    out = pl.pallas_call(
        _rpb_kernel,
        grid=(_H,),
        in_specs=[pl.BlockSpec((1, 1, 512), lambda h: (h, 0, 0))],
        out_specs=pl.BlockSpec((1, 1, _S, _S), lambda h: (0, h, 0, 0)),
        out_shape=jax.ShapeDtypeStruct((1, _H, _S, _S), jnp.float32),
        scratch_shapes=[pltpu.VMEM((128, _EXT), jnp.float32)],
        compiler_params=pltpu.CompilerParams(
            dimension_semantics=("arbitrary",)),
    )(tableT)
    return out
